# Initial kernel scaffold; baseline (speedup 1.0000x reference)
import jax
import jax.numpy as jnp
from jax.experimental import pallas as pl


def kernel(text_tokens, graph_nodes, text_mask, W1, b1, W2, b2, ln_g, ln_b, slices):
    raise NotImplementedError("write your pallas kernel here")


# speedup vs baseline: 2.3827x; 2.3827x over previous
import functools

import jax
import jax.numpy as jnp
from jax import lax
from jax.experimental import pallas as pl
from jax.experimental.pallas import tpu as pltpu
from jax.experimental.pallas import tpu_sc as plsc

_NC, _NS, _L = 2, 16, 16
_NW = _NC * _NS
_KROWS = 128


def _seg_partial_body(nodes_hbm, slices_hbm, out_hbm, sl_v, buf0, buf1,
                      part_v, sem0, sem1):
    n_rows, hg = nodes_hbm.shape
    n_seg = out_hbm.shape[1]
    cch = hg // _L
    wid = lax.axis_index("s") * _NC + lax.axis_index("c")
    stripe = n_rows // _NW
    nchunk = stripe // _KROWS
    base = wid * stripe
    pltpu.sync_copy(slices_hbm, sl_v)
    sv0 = sl_v[pl.ds(0, _L)]
    sv1 = sl_v[pl.ds(_L, _L)]

    def sget(i):
        return sv0[i] if i < _L else sv1[i - _L]
    zeros = jnp.zeros((_L,), jnp.float32)
    for b in range(n_seg):
        for cj in range(cch):
            part_v[b, pl.ds(cj * _L, _L)] = zeros
    bufs = (buf0, buf1)
    sems = (sem0, sem1)
    copies = [None, None]
    copies[0] = pltpu.async_copy(
        nodes_hbm.at[pl.ds(base, _KROWS), :], bufs[0], sems[0]
    )
    for k in range(nchunk):
        nxt = k + 1
        if nxt < nchunk:
            copies[nxt % 2] = pltpu.async_copy(
                nodes_hbm.at[pl.ds(base + nxt * _KROWS, _KROWS), :],
                bufs[nxt % 2],
                sems[nxt % 2],
            )
        copies[k % 2].wait()
        buf = bufs[k % 2]
        cbase = base + k * _KROWS
        for b in range(n_seg):
            s0 = sget(b)
            s1 = sget(b + 1)
            lo = jnp.clip(s0 - cbase, 0, _KROWS)
            hi = jnp.clip(s1 - cbase, 0, _KROWS)

            @pl.when(hi > lo)
            def _():
                def row_body(r, accs):
                    return tuple(
                        accs[cj] + buf[r, pl.ds(cj * _L, _L)]
                        for cj in range(cch)
                    )

                accs = lax.fori_loop(
                    lo, hi, row_body, (zeros,) * cch
                )
                for cj in range(cch):
                    part_v[b, pl.ds(cj * _L, _L)] = (
                        part_v[b, pl.ds(cj * _L, _L)] + accs[cj]
                    )

    pltpu.sync_copy(part_v, out_hbm.at[wid])


def _seg_partials(graph_nodes, slices):
    n_seg = slices.shape[0] - 1
    hg = graph_nodes.shape[1]
    sl_pad = jnp.zeros((32,), jnp.int32).at[: slices.shape[0]].set(
        slices.astype(jnp.int32)
    )
    mesh = plsc.VectorSubcoreMesh(
        core_axis_name="c", subcore_axis_name="s", num_cores=_NC,
        num_subcores=_NS,
    )
    f = pl.kernel(
        _seg_partial_body,
        out_type=jax.ShapeDtypeStruct((_NW, n_seg, hg), jnp.float32),
        mesh=mesh,
        scratch_types=[
            pltpu.VMEM((32,), jnp.int32),
            pltpu.VMEM((_KROWS, hg), jnp.float32),
            pltpu.VMEM((_KROWS, hg), jnp.float32),
            pltpu.VMEM((n_seg, hg), jnp.float32),
            pltpu.SemaphoreType.DMA,
            pltpu.SemaphoreType.DMA,
        ],
    )
    return f(graph_nodes, sl_pad)


def _tsum_body(x_ref, m_ref, o_ref):
    i = pl.program_id(0)
    x = x_ref[...]
    m = m_ref[...]
    o_ref[pl.ds(i, 1), :] = jnp.sum(x * m, axis=0, keepdims=True)


def _token_sum(text_tokens, text_mask):
    b, t, ht = text_tokens.shape
    x = text_tokens.reshape(b * t, ht)
    m = text_mask.reshape(b * t, 1)
    return pl.pallas_call(
        _tsum_body,
        grid=(b,),
        in_specs=[
            pl.BlockSpec((t, ht), lambda i: (i, 0)),
            pl.BlockSpec((t, 1), lambda i: (i, 0)),
        ],
        out_specs=pl.BlockSpec((b, ht), lambda i: (0, 0)),
        out_shape=jax.ShapeDtypeStruct((b, ht), jnp.float32),
    )(x, m)


def _mlp_body(ts_ref, m_ref, gp_ref, ginv_ref, w1t_ref, w1g_ref, b1_ref,
              w2_ref, b2_ref, lng_ref, lnb_ref, o_ref):
    cnt = jnp.clip(jnp.sum(m_ref[...], axis=1, keepdims=True), 1.0, None)
    tp = ts_ref[...] / cnt
    g = jnp.sum(gp_ref[...], axis=0) * ginv_ref[...]
    h = (
        jnp.dot(tp, w1t_ref[...], preferred_element_type=jnp.float32)
        + jnp.dot(g, w1g_ref[...], preferred_element_type=jnp.float32)
        + b1_ref[...]
    )
    h = h * jax.nn.sigmoid(h)
    h = jnp.dot(h, w2_ref[...], preferred_element_type=jnp.float32) + b2_ref[...]
    h = h * jax.nn.sigmoid(h)
    mu = jnp.mean(h, axis=-1, keepdims=True)
    var = jnp.mean((h - mu) ** 2, axis=-1, keepdims=True)
    o_ref[...] = (h - mu) * lax.rsqrt(var + 1e-5) * lng_ref[...] + lnb_ref[...]


def _mlp(t_sum, text_mask, g_partials, g_inv, W1, b1, W2, b2, ln_g, ln_b):
    b, ht = t_sum.shape
    g = W2.shape[-1]
    return pl.pallas_call(
        _mlp_body,
        out_shape=jax.ShapeDtypeStruct((b, g), jnp.float32),
    )(
        t_sum,
        text_mask,
        g_partials,
        g_inv,
        W1[:ht],
        W1[ht:],
        b1.reshape(1, g),
        W2,
        b2.reshape(1, g),
        ln_g.reshape(1, g),
        ln_b.reshape(1, g),
    )


def kernel(text_tokens, graph_nodes, text_mask, W1, b1, W2, b2, ln_g, ln_b,
           slices):
    g_partials = _seg_partials(graph_nodes, slices)
    t_sum = _token_sum(text_tokens, text_mask)
    counts = (slices[1:] - slices[:-1]).astype(jnp.float32)
    g_inv = jnp.where(counts > 0, 1.0 / jnp.maximum(counts, 1.0), 0.0)
    return _mlp(t_sum, text_mask, g_partials, g_inv[:, None], W1, b1, W2, b2,
                ln_g, ln_b)


# speedup vs baseline: 2.5458x; 1.0684x over previous
import functools

import jax
import jax.numpy as jnp
from jax import lax
from jax.experimental import pallas as pl
from jax.experimental.pallas import tpu as pltpu
from jax.experimental.pallas import tpu_sc as plsc

_NC, _NS, _L = 2, 16, 16
_NW = _NC * _NS
_KROWS = 128


def _seg_partial_body(nodes_hbm, slices_hbm, out_hbm, sl_v, buf0, buf1,
                      part_v, sem0, sem1):
    n_rows, hg = nodes_hbm.shape
    n_seg = out_hbm.shape[1]
    cch = hg // _L
    wid = lax.axis_index("s") * _NC + lax.axis_index("c")
    stripe = n_rows // _NW
    nchunk = stripe // _KROWS
    base = wid * stripe
    pltpu.sync_copy(slices_hbm, sl_v)
    sv0 = sl_v[pl.ds(0, _L)]
    sv1 = sl_v[pl.ds(1, _L)]

    def sget(i):
        return sv0[i] if i < _L else sv1[i - 1]
    zeros = jnp.zeros((_L,), jnp.float32)
    for b in range(n_seg):
        for cj in range(cch):
            part_v[b, pl.ds(cj * _L, _L)] = zeros
    bufs = (buf0, buf1)
    sems = (sem0, sem1)
    copies = [None, None]
    copies[0] = pltpu.async_copy(
        nodes_hbm.at[pl.ds(base, _KROWS), :], bufs[0], sems[0]
    )
    for k in range(nchunk):
        nxt = k + 1
        if nxt < nchunk:
            copies[nxt % 2] = pltpu.async_copy(
                nodes_hbm.at[pl.ds(base + nxt * _KROWS, _KROWS), :],
                bufs[nxt % 2],
                sems[nxt % 2],
            )
        copies[k % 2].wait()
        buf = bufs[k % 2]
        cbase = base + k * _KROWS
        for b in range(n_seg):
            s0 = sget(b)
            s1 = sget(b + 1)
            lo = jnp.clip(s0 - cbase, 0, _KROWS)
            hi = jnp.clip(s1 - cbase, 0, _KROWS)

            @pl.when(hi > lo)
            def _():
                def row_body(r, accs):
                    return tuple(
                        accs[cj] + buf[r, pl.ds(cj * _L, _L)]
                        for cj in range(cch)
                    )

                accs = lax.fori_loop(
                    lo, hi, row_body, (zeros,) * cch
                )
                for cj in range(cch):
                    part_v[b, pl.ds(cj * _L, _L)] = (
                        part_v[b, pl.ds(cj * _L, _L)] + accs[cj]
                    )

    pltpu.sync_copy(part_v, out_hbm.at[wid])


def _seg_partials(graph_nodes, slices):
    n_seg = slices.shape[0] - 1
    hg = graph_nodes.shape[1]
    mesh = plsc.VectorSubcoreMesh(
        core_axis_name="c", subcore_axis_name="s", num_cores=_NC,
        num_subcores=_NS,
    )
    f = pl.kernel(
        _seg_partial_body,
        out_type=jax.ShapeDtypeStruct((_NW, n_seg, hg), jnp.float32),
        mesh=mesh,
        compiler_params=pltpu.CompilerParams(use_tc_tiling_on_sc=True),
        scratch_types=[
            pltpu.VMEM((n_seg + 1,), jnp.int32),
            pltpu.VMEM((_KROWS, hg), jnp.float32),
            pltpu.VMEM((_KROWS, hg), jnp.float32),
            pltpu.VMEM((n_seg, hg), jnp.float32),
            pltpu.SemaphoreType.DMA,
            pltpu.SemaphoreType.DMA,
        ],
    )
    return f(graph_nodes, slices.astype(jnp.int32))


def _tsum_body(x_ref, m_ref, o_ref):
    i = pl.program_id(0)
    x = x_ref[...]
    m = m_ref[...]
    o_ref[pl.ds(i, 1), :] = jnp.sum(x * m, axis=0, keepdims=True)


def _token_sum(text_tokens, text_mask):
    b, t, ht = text_tokens.shape
    x = text_tokens.reshape(b * t, ht)
    m = text_mask.reshape(b * t, 1)
    return pl.pallas_call(
        _tsum_body,
        grid=(b,),
        in_specs=[
            pl.BlockSpec((t, ht), lambda i: (i, 0)),
            pl.BlockSpec((t, 1), lambda i: (i, 0)),
        ],
        out_specs=pl.BlockSpec((b, ht), lambda i: (0, 0)),
        out_shape=jax.ShapeDtypeStruct((b, ht), jnp.float32),
    )(x, m)


def _mlp_body(ts_ref, m_ref, gp_ref, ginv_ref, w1_ref, b1_ref,
              w2_ref, b2_ref, lng_ref, lnb_ref, o_ref):
    ht = ts_ref.shape[1]
    cnt = jnp.clip(jnp.sum(m_ref[...], axis=1, keepdims=True), 1.0, None)
    tp = ts_ref[...] / cnt
    g = jnp.sum(gp_ref[...], axis=0) * ginv_ref[...]
    h = (
        jnp.dot(tp, w1_ref[0:ht], preferred_element_type=jnp.float32)
        + jnp.dot(g, w1_ref[ht:], preferred_element_type=jnp.float32)
        + b1_ref[...]
    )
    h = h * jax.nn.sigmoid(h)
    h = jnp.dot(h, w2_ref[...], preferred_element_type=jnp.float32) + b2_ref[...]
    h = h * jax.nn.sigmoid(h)
    mu = jnp.mean(h, axis=-1, keepdims=True)
    var = jnp.mean((h - mu) ** 2, axis=-1, keepdims=True)
    o_ref[...] = (h - mu) * lax.rsqrt(var + 1e-5) * lng_ref[...] + lnb_ref[...]


def _mlp(t_sum, text_mask, g_partials, g_inv, W1, b1, W2, b2, ln_g, ln_b):
    b, ht = t_sum.shape
    g = W2.shape[-1]
    return pl.pallas_call(
        _mlp_body,
        out_shape=jax.ShapeDtypeStruct((b, g), jnp.float32),
    )(
        t_sum,
        text_mask,
        g_partials,
        g_inv,
        W1,
        b1.reshape(1, g),
        W2,
        b2.reshape(1, g),
        ln_g.reshape(1, g),
        ln_b.reshape(1, g),
    )


def kernel(text_tokens, graph_nodes, text_mask, W1, b1, W2, b2, ln_g, ln_b,
           slices):
    g_partials = _seg_partials(graph_nodes, slices)
    t_sum = _token_sum(text_tokens, text_mask)
    counts = (slices[1:] - slices[:-1]).astype(jnp.float32)
    g_inv = jnp.where(counts > 0, 1.0 / jnp.maximum(counts, 1.0), 0.0)
    return _mlp(t_sum, text_mask, g_partials, g_inv[:, None], W1, b1, W2, b2,
                ln_g, ln_b)


# speedup vs baseline: 3.0166x; 1.1850x over previous
import functools

import jax
import jax.numpy as jnp
from jax import lax
from jax.experimental import pallas as pl
from jax.experimental.pallas import tpu as pltpu
from jax.experimental.pallas import tpu_sc as plsc

_NC, _NS, _L = 2, 16, 16
_NW = _NC * _NS
_KROWS = 128


def _seg_partial_body(nodes_hbm, slices_hbm, out_hbm, sl_v, buf0, buf1,
                      part_v, sem0, sem1):
    n_rows, hg = nodes_hbm.shape
    n_seg = out_hbm.shape[1]
    cch = hg // _L
    wid = lax.axis_index("s") * _NC + lax.axis_index("c")
    stripe = n_rows // _NW
    nchunk = stripe // _KROWS
    base = wid * stripe
    pltpu.sync_copy(slices_hbm, sl_v)
    sv0 = sl_v[pl.ds(0, _L)]
    sv1 = sl_v[pl.ds(1, _L)]

    def sget(i):
        return sv0[i] if i < _L else sv1[i - 1]
    zeros = jnp.zeros((_L,), jnp.float32)
    for b in range(n_seg):
        for cj in range(cch):
            part_v[b, pl.ds(cj * _L, _L)] = zeros
    bufs = (buf0, buf1)
    sems = (sem0, sem1)
    copies = [None, None]
    copies[0] = pltpu.async_copy(
        nodes_hbm.at[pl.ds(base, _KROWS), :], bufs[0], sems[0]
    )
    for k in range(nchunk):
        nxt = k + 1
        if nxt < nchunk:
            copies[nxt % 2] = pltpu.async_copy(
                nodes_hbm.at[pl.ds(base + nxt * _KROWS, _KROWS), :],
                bufs[nxt % 2],
                sems[nxt % 2],
            )
        copies[k % 2].wait()
        buf = bufs[k % 2]
        cbase = base + k * _KROWS
        for b in range(n_seg):
            s0 = sget(b)
            s1 = sget(b + 1)
            lo = jnp.clip(s0 - cbase, 0, _KROWS)
            hi = jnp.clip(s1 - cbase, 0, _KROWS)

            @pl.when(hi > lo)
            def _():
                def row_body(r, accs):
                    return tuple(
                        accs[cj] + buf[r, pl.ds(cj * _L, _L)]
                        for cj in range(cch)
                    )

                accs = lax.fori_loop(
                    lo, hi, row_body, (zeros,) * cch
                )
                for cj in range(cch):
                    part_v[b, pl.ds(cj * _L, _L)] = (
                        part_v[b, pl.ds(cj * _L, _L)] + accs[cj]
                    )

    pltpu.sync_copy(part_v, out_hbm.at[wid])


def _seg_partials(graph_nodes, slices):
    n_seg = slices.shape[0] - 1
    hg = graph_nodes.shape[1]
    mesh = plsc.VectorSubcoreMesh(
        core_axis_name="c", subcore_axis_name="s", num_cores=_NC,
        num_subcores=_NS,
    )
    f = pl.kernel(
        _seg_partial_body,
        out_type=jax.ShapeDtypeStruct((_NW, n_seg, hg), jnp.float32),
        mesh=mesh,
        compiler_params=pltpu.CompilerParams(use_tc_tiling_on_sc=True),
        scratch_types=[
            pltpu.VMEM((n_seg + 1,), jnp.int32),
            pltpu.VMEM((_KROWS, hg), jnp.float32),
            pltpu.VMEM((_KROWS, hg), jnp.float32),
            pltpu.VMEM((n_seg, hg), jnp.float32),
            pltpu.SemaphoreType.DMA,
            pltpu.SemaphoreType.DMA,
        ],
    )
    return f(graph_nodes, slices.astype(jnp.int32))


def _tsum_body(x_ref, m_ref, o_ref):
    i = pl.program_id(0)
    x = x_ref[...]
    m = m_ref[pl.ds(i, 1), :]
    o_ref[pl.ds(i, 1), :] = lax.dot_general(
        m, x, (((1,), (0,)), ((), ())), preferred_element_type=jnp.float32
    )


def _token_sum(text_tokens, text_mask):
    b, t, ht = text_tokens.shape
    x = text_tokens.reshape(b * t, ht)
    return pl.pallas_call(
        _tsum_body,
        grid=(b,),
        in_specs=[
            pl.BlockSpec((t, ht), lambda i: (i, 0)),
            pl.BlockSpec((b, t), lambda i: (0, 0)),
        ],
        out_specs=pl.BlockSpec((b, ht), lambda i: (0, 0)),
        out_shape=jax.ShapeDtypeStruct((b, ht), jnp.float32),
    )(x, text_mask)


def _mlp_body(ts_ref, m_ref, gp_ref, ginv_ref, w1_ref, b1_ref,
              w2_ref, b2_ref, lng_ref, lnb_ref, o_ref):
    ht = ts_ref.shape[1]
    cnt = jnp.clip(jnp.sum(m_ref[...], axis=1, keepdims=True), 1.0, None)
    tp = ts_ref[...] / cnt
    g = jnp.sum(gp_ref[...], axis=0) * ginv_ref[...]
    h = (
        jnp.dot(tp, w1_ref[0:ht], preferred_element_type=jnp.float32)
        + jnp.dot(g, w1_ref[ht:], preferred_element_type=jnp.float32)
        + b1_ref[...]
    )
    h = h * jax.nn.sigmoid(h)
    h = jnp.dot(h, w2_ref[...], preferred_element_type=jnp.float32) + b2_ref[...]
    h = h * jax.nn.sigmoid(h)
    mu = jnp.mean(h, axis=-1, keepdims=True)
    var = jnp.mean((h - mu) ** 2, axis=-1, keepdims=True)
    o_ref[...] = (h - mu) * lax.rsqrt(var + 1e-5) * lng_ref[...] + lnb_ref[...]


def _mlp(t_sum, text_mask, g_partials, g_inv, W1, b1, W2, b2, ln_g, ln_b):
    b, ht = t_sum.shape
    g = W2.shape[-1]
    return pl.pallas_call(
        _mlp_body,
        out_shape=jax.ShapeDtypeStruct((b, g), jnp.float32),
    )(
        t_sum,
        text_mask,
        g_partials,
        g_inv,
        W1,
        b1.reshape(1, g),
        W2,
        b2.reshape(1, g),
        ln_g.reshape(1, g),
        ln_b.reshape(1, g),
    )


def kernel(text_tokens, graph_nodes, text_mask, W1, b1, W2, b2, ln_g, ln_b,
           slices):
    g_partials = _seg_partials(graph_nodes, slices)
    t_sum = _token_sum(text_tokens, text_mask)
    counts = (slices[1:] - slices[:-1]).astype(jnp.float32)
    g_inv = jnp.where(counts > 0, 1.0 / jnp.maximum(counts, 1.0), 0.0)
    return _mlp(t_sum, text_mask, g_partials, g_inv[:, None], W1, b1, W2, b2,
                ln_g, ln_b)
